# baseline (speedup 1.0000x reference)
import jax
import jax.numpy as jnp
from jax.experimental import pallas as pl

NUM_EXPERT = 64
IN_FEAT = 768
OUT_FEAT = 768
N_TOKENS = 128
NSPLIT = 4
OSPLIT = OUT_FEAT // NSPLIT


def _moe_kernel(gate_ref, inp_ref, w0, w1, w2, w3, out_ref):
    e = pl.program_id(0)

    @pl.when(e == 0)
    def _init():
        out_ref[...] = jnp.zeros_like(out_ref)

    mask = gate_ref[...] == e
    x = jnp.where(mask, inp_ref[...], 0.0)
    for k, w in enumerate((w0, w1, w2, w3)):
        partial = jax.lax.dot_general(
            x, w[0],
            (((1,), (1,)), ((), ())),
            preferred_element_type=jnp.float32,
        )
        out_ref[:, k * OSPLIT:(k + 1) * OSPLIT] += partial


def kernel(inp, gate, weight):
    gate2d = gate.reshape(N_TOKENS, 1)
    w_spec = [
        pl.BlockSpec((1, OSPLIT, IN_FEAT), lambda e, kk=k: (e, kk, 0))
        for k in range(NSPLIT)
    ]
    return pl.pallas_call(
        _moe_kernel,
        grid=(NUM_EXPERT,),
        in_specs=[
            pl.BlockSpec((N_TOKENS, 1), lambda e: (0, 0)),
            pl.BlockSpec((N_TOKENS, IN_FEAT), lambda e: (0, 0)),
        ] + w_spec,
        out_specs=pl.BlockSpec((N_TOKENS, OUT_FEAT), lambda e: (0, 0)),
        out_shape=jax.ShapeDtypeStruct((N_TOKENS, OUT_FEAT), jnp.float32),
    )(gate2d, inp, weight, weight, weight, weight)


# speedup vs baseline: 1.6216x; 1.6216x over previous
import jax
import jax.numpy as jnp
from jax.experimental import pallas as pl

NUM_EXPERT = 64
IN_FEAT = 768
OUT_FEAT = 768
N_TOKENS = 128
EPG = 4
NSTEPS = NUM_EXPERT // EPG


def _moe_kernel(gate_ref, inp_ref, w_ref, out_ref):
    s = pl.program_id(0)

    @pl.when(s == 0)
    def _init():
        out_ref[...] = jnp.zeros_like(out_ref)

    acc = out_ref[...]
    for j in range(EPG):
        e = s * EPG + j
        mask = gate_ref[...] == e
        x = jnp.where(mask, inp_ref[...], 0.0)
        acc += jax.lax.dot_general(
            x, w_ref[j],
            (((1,), (1,)), ((), ())),
            preferred_element_type=jnp.float32,
        )
    out_ref[...] = acc


def kernel(inp, gate, weight):
    gate2d = gate.reshape(N_TOKENS, 1)
    return pl.pallas_call(
        _moe_kernel,
        grid=(NSTEPS,),
        in_specs=[
            pl.BlockSpec((N_TOKENS, 1), lambda s: (0, 0)),
            pl.BlockSpec((N_TOKENS, IN_FEAT), lambda s: (0, 0)),
            pl.BlockSpec((EPG, OUT_FEAT, IN_FEAT), lambda s: (s, 0, 0)),
        ],
        out_specs=pl.BlockSpec((N_TOKENS, OUT_FEAT), lambda s: (0, 0)),
        out_shape=jax.ShapeDtypeStruct((N_TOKENS, OUT_FEAT), jnp.float32),
    )(gate2d, inp, weight)
